# baseline (speedup 1.0000x reference)
import functools

import numpy as np
import jax
import jax.numpy as jnp
from jax import lax
from jax.experimental import pallas as pl
from jax.experimental.pallas import tpu as pltpu
from jax.experimental.pallas import tpu_sc as plsc

SEQ_LEN = 200
BATCH = 4096
VOCAB = 100000
DIM = 64

_info = plsc.get_sparse_core_info()
NC = _info.num_cores
NS = _info.num_subcores
NW = NC * NS
BATCH_W = BATCH // NW
SPLITS = ((0, 128), (128, 72))


def _pe_last_position() -> np.ndarray:
    pos = SEQ_LEN - 1
    pe = np.zeros(DIM)
    for i in range(DIM):
        if i % 2 == 0:
            pe[i] = np.sin(pos / 10000 ** (i / DIM))
        else:
            pe[i] = np.cos(pos / 10000 ** ((i - 1) / DIM))
    return pe.astype(np.float32)


_PE8 = np.tile(_pe_last_position()[None, :], (8, 1))


def _pe_add_body(t_ref, pe_ref, o_ref):
    x = t_ref[...] + pe_ref[0:1, :]
    o_ref[:, 0:DIM] = x
    o_ref[:, DIM:2 * DIM] = x


def _add_pe_dup(table):
    nblk = 10
    rows = VOCAB // nblk
    return pl.pallas_call(
        _pe_add_body,
        grid=(nblk,),
        in_specs=[
            pl.BlockSpec((rows, DIM), lambda i: (i, 0)),
            pl.BlockSpec((8, DIM), lambda i: (0, 0)),
        ],
        out_specs=pl.BlockSpec((rows, 2 * DIM), lambda i: (i, 0)),
        out_shape=jax.ShapeDtypeStruct((VOCAB, 2 * DIM), jnp.float32),
    )(table, jnp.asarray(_PE8))


NB = 2
HALF_W = BATCH_W // 2
NGRP = HALF_W // NB


def _gathers_desc(table_hbm, idx_v, buf, sem, g):
    for bb in range(NB):
        s = NB * g + bb
        for h, (off, n) in enumerate(SPLITS):
            yield pltpu.make_async_copy(
                table_hbm.at[idx_v.at[2 * s + h, pl.ds(0, n)]],
                buf.at[bb, pl.ds(off, n)],
                sem,
            )


def _copyout_desc(out_hbm, buf, sem, bbase, g):
    return pltpu.make_async_copy(buf, out_hbm.at[pl.ds(bbase + NB * g, NB)], sem)


@functools.partial(
    pl.kernel,
    mesh=plsc.VectorSubcoreMesh(core_axis_name="c", subcore_axis_name="s"),
    out_type=jax.ShapeDtypeStruct((BATCH, SEQ_LEN, 2 * DIM), jnp.float32),
    scratch_types=[
        pltpu.VMEM((2 * HALF_W, 128), jnp.int32),
        pltpu.VMEM((NB, SEQ_LEN, 2 * DIM), jnp.float32),
        pltpu.VMEM((NB, SEQ_LEN, 2 * DIM), jnp.float32),
        pltpu.SemaphoreType.DMA,
        pltpu.SemaphoreType.DMA,
        pltpu.SemaphoreType.DMA,
        pltpu.SemaphoreType.DMA,
    ],
    compiler_params=pltpu.CompilerParams(use_tc_tiling_on_sc=True),
)
def _gather(table_hbm, idx_hbm, out_hbm, idx_v, buf_a, buf_b, sga, sgb, soa, sob):
    wid = lax.axis_index("s") * NC + lax.axis_index("c")

    def fire_gathers(buf, sem, g):
        for d in _gathers_desc(table_hbm, idx_v, buf, sem, g):
            d.start()

    def wait_gathers(buf, sem, g):
        for d in _gathers_desc(table_hbm, idx_v, buf, sem, g):
            d.wait()

    for p in range(2):
        bbase = wid * BATCH_W + p * HALF_W
        pltpu.sync_copy(
            idx_hbm.at[pl.ds((wid * 2 + p) * 2 * HALF_W, 2 * HALF_W)], idx_v
        )

        fire_gathers(buf_a, sga, 0)

        def step(t, carry):
            ga = 2 * t
            gb = 2 * t + 1

            @pl.when(t > 0)
            def _():
                _copyout_desc(out_hbm, buf_b, sob, bbase, gb - 2).wait()

            fire_gathers(buf_b, sgb, gb)
            wait_gathers(buf_a, sga, ga)
            _copyout_desc(out_hbm, buf_a, soa, bbase, ga).start()
            _copyout_desc(out_hbm, buf_a, soa, bbase, ga).wait()

            @pl.when(t < NGRP // 2 - 1)
            def _():
                fire_gathers(buf_a, sga, ga + 2)

            wait_gathers(buf_b, sgb, gb)
            _copyout_desc(out_hbm, buf_b, sob, bbase, gb).start()
            return carry

        lax.fori_loop(0, NGRP // 2, step, 0)
        _copyout_desc(out_hbm, buf_b, sob, bbase, NGRP - 1).wait()


def kernel(inputs, table):
    table_dup = _add_pe_dup(table)
    idx = jnp.pad(inputs, ((0, 0), (0, 256 - SEQ_LEN))).reshape(2 * BATCH, 128)
    return _gather(table_dup, idx)[:, :, :DIM]


# speedup vs baseline: 1.0071x; 1.0071x over previous
import functools

import numpy as np
import jax
import jax.numpy as jnp
from jax import lax
from jax.experimental import pallas as pl
from jax.experimental.pallas import tpu as pltpu
from jax.experimental.pallas import tpu_sc as plsc

SEQ_LEN = 200
BATCH = 4096
VOCAB = 100000
DIM = 64

_info = plsc.get_sparse_core_info()
NC = _info.num_cores
NS = _info.num_subcores
NW = NC * NS
BATCH_W = BATCH // NW
SPLITS = ((0, 128), (128, 72))


def _pe_last_position() -> np.ndarray:
    pos = SEQ_LEN - 1
    pe = np.zeros(DIM)
    for i in range(DIM):
        if i % 2 == 0:
            pe[i] = np.sin(pos / 10000 ** (i / DIM))
        else:
            pe[i] = np.cos(pos / 10000 ** ((i - 1) / DIM))
    return pe.astype(np.float32)


_PE8 = np.tile(_pe_last_position()[None, :], (8, 1))


def _pe_add_body(t_ref, pe_ref, o_ref):
    o_ref[:, 0:DIM] = t_ref[...] + pe_ref[0:1, :]


def _add_pe_dup(table):
    nblk = 10
    rows = VOCAB // nblk
    return pl.pallas_call(
        _pe_add_body,
        grid=(nblk,),
        in_specs=[
            pl.BlockSpec((rows, DIM), lambda i: (i, 0)),
            pl.BlockSpec((8, DIM), lambda i: (0, 0)),
        ],
        out_specs=pl.BlockSpec((rows, 2 * DIM), lambda i: (i, 0)),
        out_shape=jax.ShapeDtypeStruct((VOCAB, 2 * DIM), jnp.float32),
    )(table, jnp.asarray(_PE8))


NPASS = 1
NB = 1
HALF_W = BATCH_W // NPASS
NGRP = HALF_W // NB


def _gathers_desc(table_hbm, idx_v, buf, sem, g):
    for bb in range(NB):
        s = NB * g + bb
        for h, (off, n) in enumerate(SPLITS):
            yield pltpu.make_async_copy(
                table_hbm.at[idx_v.at[2 * s + h, pl.ds(0, n)]],
                buf.at[bb, pl.ds(off, n)],
                sem,
            )


def _copyout_desc(out_hbm, buf, sem, bbase, g):
    return pltpu.make_async_copy(buf, out_hbm.at[pl.ds(bbase + NB * g, NB)], sem)


@functools.partial(
    pl.kernel,
    mesh=plsc.VectorSubcoreMesh(core_axis_name="c", subcore_axis_name="s"),
    out_type=jax.ShapeDtypeStruct((BATCH, SEQ_LEN, 2 * DIM), jnp.float32),
    scratch_types=[
        pltpu.VMEM((2 * HALF_W, 128), jnp.int32),
        pltpu.VMEM((NB, SEQ_LEN, 2 * DIM), jnp.float32),
        pltpu.VMEM((NB, SEQ_LEN, 2 * DIM), jnp.float32),
        pltpu.SemaphoreType.DMA,
        pltpu.SemaphoreType.DMA,
        pltpu.SemaphoreType.DMA,
        pltpu.SemaphoreType.DMA,
    ],
    compiler_params=pltpu.CompilerParams(use_tc_tiling_on_sc=True),
)
def _gather(table_hbm, idx_hbm, out_hbm, idx_v, buf_a, buf_b, sga, sgb, soa, sob):
    wid = lax.axis_index("s") * NC + lax.axis_index("c")

    def fire_gathers(buf, sem, g):
        for d in _gathers_desc(table_hbm, idx_v, buf, sem, g):
            d.start()

    def wait_gathers(buf, sem, g):
        for d in _gathers_desc(table_hbm, idx_v, buf, sem, g):
            d.wait()

    for p in range(NPASS):
        bbase = wid * BATCH_W + p * HALF_W
        pltpu.sync_copy(
            idx_hbm.at[pl.ds(2 * (wid * BATCH_W + p * HALF_W), 2 * HALF_W)], idx_v
        )

        fire_gathers(buf_a, sga, 0)

        def step(t, carry):
            ga = 2 * t
            gb = 2 * t + 1

            @pl.when(t > 0)
            def _():
                _copyout_desc(out_hbm, buf_b, sob, bbase, gb - 2).wait()

            fire_gathers(buf_b, sgb, gb)
            wait_gathers(buf_a, sga, ga)
            _copyout_desc(out_hbm, buf_a, soa, bbase, ga).start()
            _copyout_desc(out_hbm, buf_a, soa, bbase, ga).wait()

            @pl.when(t < NGRP // 2 - 1)
            def _():
                fire_gathers(buf_a, sga, ga + 2)

            wait_gathers(buf_b, sgb, gb)
            _copyout_desc(out_hbm, buf_b, sob, bbase, gb).start()
            return carry

        lax.fori_loop(0, NGRP // 2, step, 0)
        _copyout_desc(out_hbm, buf_b, sob, bbase, NGRP - 1).wait()


def kernel(inputs, table):
    table_dup = _add_pe_dup(table)
    idx = jnp.pad(inputs, ((0, 0), (0, 256 - SEQ_LEN))).reshape(2 * BATCH, 128)
    return _gather(table_dup, idx)[:, :, :DIM]
